# baseline (speedup 1.0000x reference)
import functools

import jax
import jax.numpy as jnp
import numpy as np
from jax import lax
from jax.experimental import pallas as pl
from jax.experimental.pallas import tpu as pltpu
from jax.experimental.pallas import tpu_sc as plsc

_MASK_RATIO = 0.75
_INT_MIN = np.int32(-2147483648)
_L = 16


def _row_select_body(bits_hbm, out_hbm, key_v, out_v, hist_v, *,
                     n_rows, hw, len_keep):
    nv = hw // _L
    wid = lax.axis_index("s") * 2 + lax.axis_index("c")

    @pl.when(wid < n_rows)
    def _():
        row = wid
        pltpu.sync_copy(bits_hbm.at[row], key_v)

        lane = lax.iota(jnp.int32, _L)
        lane_base = lane * np.int32(256)
        ones = jnp.full((_L,), 1, jnp.int32)
        zeros16 = jnp.zeros((_L,), jnp.int32)

        def zero_hist(j, _):
            hist_v[pl.ds(j * _L, _L)] = zeros16
            return 0

        def hist_pass(p, carry):
            prefix, rem = carry
            lax.fori_loop(0, 256, zero_hist, 0, unroll=8)
            shift = 24 - 8 * p

            def scan(i, _):
                v = key_v[pl.ds(i * _L, _L)]
                if p == 0:
                    v = jnp.where(v == _INT_MIN, np.int32(0), v)
                    v = v ^ ((v >> 31) & np.int32(0x7FFFFFFF))
                    key_v[pl.ds(i * _L, _L)] = v
                ku = v ^ _INT_MIN
                bucket = lax.shift_right_logical(ku, shift) & np.int32(0xFF)
                idx = lane_base + bucket
                if p == 0:
                    plsc.addupdate_scatter(hist_v, [idx], ones)
                else:
                    active = lax.shift_right_logical(ku, shift + 8) == prefix
                    plsc.addupdate_scatter(hist_v, [idx], ones, mask=active)
                return 0

            lax.fori_loop(0, nv, scan, 0)

            def select(j, sc):
                nlt, below, off = sc
                acc = hist_v[pl.ds(j * _L, _L)]
                for l in range(1, _L):
                    acc = acc + hist_v[pl.ds(l * 256 + j * _L, _L)]
                cum = off + plsc.cumsum(acc)
                m = cum < rem
                nlt = nlt + jnp.sum(m.astype(jnp.int32))
                below = below + jnp.sum(jnp.where(m, acc, 0))
                off = off + jnp.sum(acc)
                return nlt, below, off

            bkt, below, _ = lax.fori_loop(0, 16, select,
                                          (jnp.int32(0), jnp.int32(0),
                                           jnp.int32(0)))
            return (prefix << 8) | bkt, rem - below

        prefix = jnp.int32(0)
        rem = jnp.int32(len_keep)
        for p in range(4):
            prefix, rem = hist_pass(p, (prefix, rem))

        t_ks = prefix ^ _INT_MIN

        def emit(i, running):
            v = key_v[pl.ds(i * _L, _L)]
            eq = v == t_ks
            eqi = eq.astype(jnp.int32)
            cume = plsc.cumsum(eqi) + running
            zero = (v < t_ks) | (eq & (cume <= rem))
            out_v[pl.ds(i * _L, _L)] = jnp.where(zero, 0.0, 1.0)
            return running + jnp.sum(eqi)

        lax.fori_loop(0, nv, emit, jnp.int32(0))
        pltpu.sync_copy(out_v, out_hbm.at[row])


def kernel(image, importance):
    n, c, h, w = image.shape
    hw = h * w
    len_keep = int(hw * (1 - _MASK_RATIO))
    bits = lax.bitcast_convert_type(importance.reshape(n, hw), jnp.int32)

    body = functools.partial(_row_select_body, n_rows=n, hw=hw,
                             len_keep=len_keep)
    mask = pl.kernel(
        body,
        out_type=jax.ShapeDtypeStruct((n, hw), jnp.float32),
        mesh=plsc.VectorSubcoreMesh(core_axis_name="c", subcore_axis_name="s"),
        compiler_params=pltpu.CompilerParams(needs_layout_passes=False),
        scratch_types=[
            pltpu.VMEM((hw,), jnp.int32),
            pltpu.VMEM((hw,), jnp.float32),
            pltpu.VMEM((_L * 256,), jnp.int32),
        ],
    )(bits)
    return mask.reshape(n, 1, h, w)
